# baseline (speedup 1.0000x reference)
import functools

import jax
import jax.numpy as jnp
from jax import lax
from jax.experimental import pallas as pl
from jax.experimental.pallas import tpu as pltpu
from jax.experimental.pallas import tpu_sc as plsc

NUM_CORES = 2
NUM_SUBCORES = 16
LANES = 16
NW = NUM_CORES * NUM_SUBCORES

BATCH_SIZE = 16384
DIM = 32
PER_W = BATCH_SIZE // NW
CHUNK = 128
NCHUNK = PER_W // CHUNK
GROUPS = PER_W // LANES


def _tec_body(h_hbm, r_hbm, t_hbm, ent_hbm, rel_hbm, out_hbm,
              idx_h, idx_r, idx_t, h_rows, r_rows, t_rows, out_v, sem):
    wid = lax.axis_index("s") * NUM_CORES + lax.axis_index("c")
    row0 = wid * NCHUNK

    pltpu.sync_copy(h_hbm.at[pl.ds(row0, NCHUNK)], idx_h)
    pltpu.sync_copy(r_hbm.at[pl.ds(row0, NCHUNK)], idx_r)
    pltpu.sync_copy(t_hbm.at[pl.ds(row0, NCHUNK)], idx_t)

    copies = []
    for c in range(NCHUNK):
        copies.append(pltpu.async_copy(
            ent_hbm.at[idx_h.at[c]], h_rows.at[pl.ds(c * CHUNK, CHUNK)], sem))
        copies.append(pltpu.async_copy(
            rel_hbm.at[idx_r.at[c]], r_rows.at[pl.ds(c * CHUNK, CHUNK)], sem))
        copies.append(pltpu.async_copy(
            ent_hbm.at[idx_t.at[c]], t_rows.at[pl.ds(c * CHUNK, CHUNK)], sem))
    for cp in copies:
        cp.wait()

    lane = lax.iota(jnp.int32, LANES)

    def group(g, _):
        acc = jnp.zeros((LANES,), jnp.float32)
        for j in range(LANES):
            i = g * LANES + j
            e0 = jnp.abs(h_rows[i, pl.ds(0, LANES)]
                         + r_rows[i, pl.ds(0, LANES)]
                         - t_rows[i, pl.ds(0, LANES)])
            e1 = jnp.abs(h_rows[i, pl.ds(LANES, LANES)]
                         + r_rows[i, pl.ds(LANES, LANES)]
                         - t_rows[i, pl.ds(LANES, LANES)])
            total = jnp.sum(e0 + e1)
            acc = jnp.where(lane == j, total, acc)
        out_v[pl.ds(g * LANES, LANES)] = acc
        return 0

    lax.fori_loop(0, GROUPS, group, 0)

    pltpu.sync_copy(out_v, out_hbm.at[pl.ds(wid * PER_W, PER_W)])


@jax.jit
def _transe(h2, r2, t2, entity_emb, relation_emb):
    mesh = plsc.VectorSubcoreMesh(core_axis_name="c", subcore_axis_name="s",
                                  num_cores=NUM_CORES,
                                  num_subcores=NUM_SUBCORES)
    return pl.kernel(
        _tec_body,
        out_type=jax.ShapeDtypeStruct((BATCH_SIZE,), jnp.float32),
        mesh=mesh,
        scratch_types=[
            pltpu.VMEM((NCHUNK, CHUNK), jnp.int32),
            pltpu.VMEM((NCHUNK, CHUNK), jnp.int32),
            pltpu.VMEM((NCHUNK, CHUNK), jnp.int32),
            pltpu.VMEM((PER_W, DIM), jnp.float32),
            pltpu.VMEM((PER_W, DIM), jnp.float32),
            pltpu.VMEM((PER_W, DIM), jnp.float32),
            pltpu.VMEM((PER_W,), jnp.float32),
            pltpu.SemaphoreType.DMA,
        ],
        compiler_params=pltpu.CompilerParams(needs_layout_passes=False,
                                             use_tc_tiling_on_sc=False),
    )(h2, r2, t2, entity_emb, relation_emb)


def kernel(h, r, t, entity_emb, relation_emb):
    h2 = h.reshape(BATCH_SIZE // CHUNK, CHUNK).astype(jnp.int32)
    r2 = r.reshape(BATCH_SIZE // CHUNK, CHUNK).astype(jnp.int32)
    t2 = t.reshape(BATCH_SIZE // CHUNK, CHUNK).astype(jnp.int32)
    return _transe(h2, r2, t2, entity_emb, relation_emb)
